# baseline (speedup 1.0000x reference)
import math

import jax
import jax.numpy as jnp
from jax import lax
from jax.experimental import pallas as pl
from jax.experimental.pallas import tpu as pltpu

TEMPERATURE = 0.5
SPARSITY_THRESHOLD = 0.01
EDGE_WEIGHT = 0.1
LN_EPS = 1e-5


def _layer_norm(x, w, b):
    mean = jnp.mean(x, axis=-1, keepdims=True)
    m2 = jnp.mean(x * x, axis=-1, keepdims=True)
    var = m2 - mean * mean
    return (x - mean) * lax.rsqrt(var + LN_EPS) * w + b


def _gnn_chain(e, w1_ref, w2_ref, w3_ref, lnw, lnb):
    BB, S, D = e.shape
    H = w1_ref.shape[1]
    N = BB * S
    bf16 = jnp.bfloat16

    nsq = jnp.sum(e * e, axis=-1, keepdims=True)
    en = e * lax.rsqrt(jnp.maximum(nsq, 1e-12))

    n2 = jnp.sum(en * en, axis=-1, keepdims=True)
    dots = jnp.einsum('bsd,btd->bst', en, en,
                      preferred_element_type=jnp.float32)
    n2b = jnp.broadcast_to(n2, (BB, S, S))
    sq = n2b + jnp.transpose(n2b, (0, 2, 1)) - 2.0 * dots
    dist = jnp.sqrt(jnp.maximum(sq, 1e-12))
    row = lax.broadcasted_iota(jnp.int32, (BB, S, S), 1)
    col = lax.broadcasted_iota(jnp.int32, (BB, S, S), 2)
    off_diag = row != col
    dist = jnp.where(off_diag, dist, 0.0)
    scaled = dist * (-1.0 / TEMPERATURE)
    m = jnp.max(scaled, axis=-1, keepdims=True)
    p = jnp.exp(scaled - m)
    denom = jnp.sum(p, axis=-1, keepdims=True)
    attn = p * pl.reciprocal(denom, approx=True)

    A = jnp.where((attn > SPARSITY_THRESHOLD) & off_diag,
                  attn * EDGE_WEIGHT, 0.0)
    At = jnp.transpose(A, (0, 2, 1)).astype(bf16)

    def message_pass(x_flat, w_ref):
        h = jnp.dot(x_flat.astype(bf16), w_ref[...],
                    preferred_element_type=jnp.float32)
        h3 = h.reshape(BB, S, h.shape[-1]).astype(bf16)
        agg = jnp.einsum('bds,bsh->bdh', At, h3,
                         preferred_element_type=jnp.float32)
        return agg.reshape(N, h.shape[-1])

    x = en.reshape(N, D)
    x = _layer_norm(message_pass(x, w1_ref), lnw, lnb)
    x = _layer_norm(x + message_pass(x, w2_ref), lnw, lnb)
    x = _layer_norm(x + message_pass(x, w3_ref), lnw, lnb)
    return x.astype(bf16)


def _fused_gnn_kernel(tok_ref, table_ref, spiral_ref, w1_ref, w2_ref, w3_ref,
                      lnw_ref, lnb_ref, wout_ref, bout_ref, out_ref,
                      grow_ref, x_ref):
    BB, S, Vt = out_ref.shape
    D = spiral_ref.shape[2]
    blk = pl.program_id(0)

    @pl.when(pl.program_id(1) == 0)
    def _():
        iota8 = lax.broadcasted_iota(jnp.int32, (8, D), 0)
        for b in range(BB):
            for k in range(S // 8):
                tile = jnp.zeros((8, D), jnp.float32)
                for j in range(8):
                    idx = tok_ref[blk * BB + b, k * 8 + j]
                    r = table_ref[idx]
                    tile = jnp.where(iota8 == j, r, tile)
                grow_ref[b * S + k * 8:b * S + k * 8 + 8, :] = tile

        e = grow_ref[...].reshape(BB, S, D) + spiral_ref[...]
        x_ref[...] = _gnn_chain(e, w1_ref, w2_ref, w3_ref,
                                lnw_ref[...], lnb_ref[...])

    logits = jnp.dot(x_ref[...], wout_ref[...],
                     preferred_element_type=jnp.float32) + bout_ref[...]
    out_ref[...] = logits.reshape(BB, S, Vt)


def kernel(tokens, token_embedding, w1, w2, w3, ln_w, ln_b, w_out, b_out):
    B, S = tokens.shape
    V, D = token_embedding.shape
    H = w1.shape[1]
    Vout = w_out.shape[1]
    max_seq_len = 512
    num_batch_blocks = 32
    BB = B // num_batch_blocks

    pos = jnp.arange(S, dtype=jnp.float32)
    thetas = 2.0 * math.pi * (pos / max_seq_len)
    rs = 0.3 + 0.6 * (pos / max(1, max_seq_len - 1))
    spiral = jnp.stack([rs * jnp.cos(thetas), rs * jnp.sin(thetas)], axis=-1)
    spiral_pad = jnp.concatenate(
        [spiral, jnp.zeros((S, D - 2), jnp.float32)], axis=-1)[None]
    table_shift = jnp.concatenate(
        [jnp.zeros((V, 2), jnp.float32), token_embedding[:, : D - 2]],
        axis=-1).reshape(V, 1, D)

    w1b = w1.astype(jnp.bfloat16)
    w2b = w2.astype(jnp.bfloat16)
    w3b = w3.astype(jnp.bfloat16)
    woutb = w_out.astype(jnp.bfloat16)

    num_v_tiles = 4
    Vt = Vout // num_v_tiles
    return pl.pallas_call(
        _fused_gnn_kernel,
        out_shape=jax.ShapeDtypeStruct((B, S, Vout), jnp.float32),
        grid_spec=pltpu.PrefetchScalarGridSpec(
            num_scalar_prefetch=1,
            grid=(num_batch_blocks, num_v_tiles),
            in_specs=[
                pl.BlockSpec((V, 1, D), lambda b, v, tok: (0, 0, 0)),
                pl.BlockSpec((1, S, D), lambda b, v, tok: (0, 0, 0)),
                pl.BlockSpec((D, H), lambda b, v, tok: (0, 0)),
                pl.BlockSpec((H, H), lambda b, v, tok: (0, 0)),
                pl.BlockSpec((H, H), lambda b, v, tok: (0, 0)),
                pl.BlockSpec((1, H), lambda b, v, tok: (0, 0)),
                pl.BlockSpec((1, H), lambda b, v, tok: (0, 0)),
                pl.BlockSpec((H, Vt), lambda b, v, tok: (0, v)),
                pl.BlockSpec((1, Vt), lambda b, v, tok: (0, v)),
            ],
            out_specs=pl.BlockSpec((BB, S, Vt), lambda b, v, tok: (b, 0, v)),
            scratch_shapes=[pltpu.VMEM((BB * S, D), jnp.float32),
                            pltpu.VMEM((BB * S, H), jnp.bfloat16)],
        ),
        compiler_params=pltpu.CompilerParams(
            dimension_semantics=("parallel", "arbitrary")),
    )(tokens, table_shift, spiral_pad, w1b, w2b, w3b,
      ln_w, ln_b, woutb, b_out)


# speedup vs baseline: 1.1477x; 1.1477x over previous
import math

import jax
import jax.numpy as jnp
from jax import lax
from jax.experimental import pallas as pl
from jax.experimental.pallas import tpu as pltpu

TEMPERATURE = 0.5
SPARSITY_THRESHOLD = 0.01
EDGE_WEIGHT = 0.1
LN_EPS = 1e-5


def _layer_norm(x, w, b):
    mean = jnp.mean(x, axis=-1, keepdims=True)
    m2 = jnp.mean(x * x, axis=-1, keepdims=True)
    var = m2 - mean * mean
    return (x - mean) * lax.rsqrt(var + LN_EPS) * w + b


def _gnn_chain(e, w1_ref, w2_ref, w3_ref, lnw, lnb, wout_ref, bout):
    BB, S, D = e.shape
    H = w1_ref.shape[1]
    V = wout_ref.shape[1]
    N = BB * S
    bf16 = jnp.bfloat16

    nsq = jnp.sum(e * e, axis=-1, keepdims=True)
    en = e * lax.rsqrt(jnp.maximum(nsq, 1e-12))

    dots = jnp.einsum('bsd,btd->bst', en, en,
                      preferred_element_type=jnp.float32)
    sq = 2.0 - 2.0 * dots
    dist = jnp.sqrt(jnp.maximum(sq, 1e-12))
    row = lax.broadcasted_iota(jnp.int32, (BB, S, S), 1)
    col = lax.broadcasted_iota(jnp.int32, (BB, S, S), 2)
    off_diag = row != col
    dist = jnp.where(off_diag, dist, 0.0)
    p = jnp.exp(dist * (-1.0 / TEMPERATURE))
    denom = jnp.sum(p, axis=-1, keepdims=True)
    attn = p * pl.reciprocal(denom, approx=True)

    A = jnp.where((attn > SPARSITY_THRESHOLD) & off_diag,
                  attn * EDGE_WEIGHT, 0.0)
    At = jnp.transpose(A, (0, 2, 1)).astype(bf16)

    def message_pass(x_flat, w_ref):
        h = jnp.dot(x_flat.astype(bf16), w_ref[...],
                    preferred_element_type=jnp.float32)
        h3 = h.reshape(BB, S, h.shape[-1]).astype(bf16)
        agg = jnp.einsum('bds,bsh->bdh', At, h3,
                         preferred_element_type=jnp.float32)
        return agg.reshape(N, h.shape[-1])

    x = en.reshape(N, D)
    x = _layer_norm(message_pass(x, w1_ref), lnw, lnb)
    x = _layer_norm(x + message_pass(x, w2_ref), lnw, lnb)
    x = _layer_norm(x + message_pass(x, w3_ref), lnw, lnb)

    logits = jnp.dot(x.astype(bf16), wout_ref[...],
                     preferred_element_type=jnp.float32) + bout
    return logits.reshape(BB, S, V)


def _fused_gnn_kernel(tok_ref, table_ref, spiral_ref, w1_ref, w2_ref, w3_ref,
                      lnw_ref, lnb_ref, wout_ref, bout_ref, out_ref,
                      grow_ref):
    BB, S, Vout = out_ref.shape
    D = spiral_ref.shape[2]
    blk = pl.program_id(0)
    iota8 = lax.broadcasted_iota(jnp.int32, (8, D), 0)

    for b in range(BB):
        for k in range(S // 8):
            tile = spiral_ref[0, k * 8:k * 8 + 8, :]
            for j in range(8):
                idx = tok_ref[blk * BB + b, k * 8 + j]
                r = table_ref[idx]
                tile = jnp.where(iota8 == j, r + tile, tile)
            grow_ref[b * S + k * 8:b * S + k * 8 + 8, :] = tile

    e = grow_ref[...].reshape(BB, S, D)

    lnw = lnw_ref[...]
    lnb = lnb_ref[...]
    bout = bout_ref[...]
    out_ref[...] = _gnn_chain(e, w1_ref, w2_ref, w3_ref,
                              lnw, lnb, wout_ref, bout)


def kernel(tokens, token_embedding, w1, w2, w3, ln_w, ln_b, w_out, b_out):
    B, S = tokens.shape
    V, D = token_embedding.shape
    H = w1.shape[1]
    Vout = w_out.shape[1]
    max_seq_len = 512
    num_batch_blocks = 32
    BB = B // num_batch_blocks

    pos = jnp.arange(S, dtype=jnp.float32)
    thetas = 2.0 * math.pi * (pos / max_seq_len)
    rs = 0.3 + 0.6 * (pos / max(1, max_seq_len - 1))
    spiral = jnp.stack([rs * jnp.cos(thetas), rs * jnp.sin(thetas)], axis=-1)
    spiral_pad = jnp.concatenate(
        [spiral, jnp.zeros((S, D - 2), jnp.float32)], axis=-1)[None]
    table_shift = jnp.concatenate(
        [jnp.zeros((V, 2), jnp.float32), token_embedding[:, : D - 2]],
        axis=-1).reshape(V, 1, D)

    w1b = w1.astype(jnp.bfloat16)
    w2b = w2.astype(jnp.bfloat16)
    w3b = w3.astype(jnp.bfloat16)
    woutb = w_out.astype(jnp.bfloat16)

    return pl.pallas_call(
        _fused_gnn_kernel,
        out_shape=jax.ShapeDtypeStruct((B, S, Vout), jnp.float32),
        grid_spec=pltpu.PrefetchScalarGridSpec(
            num_scalar_prefetch=1,
            grid=(num_batch_blocks,),
            in_specs=[
                pl.BlockSpec((V, 1, D), lambda b, tok: (0, 0, 0)),
                pl.BlockSpec((1, S, D), lambda b, tok: (0, 0, 0)),
                pl.BlockSpec((D, H), lambda b, tok: (0, 0)),
                pl.BlockSpec((H, H), lambda b, tok: (0, 0)),
                pl.BlockSpec((H, H), lambda b, tok: (0, 0)),
                pl.BlockSpec((1, H), lambda b, tok: (0, 0)),
                pl.BlockSpec((1, H), lambda b, tok: (0, 0)),
                pl.BlockSpec((H, Vout), lambda b, tok: (0, 0)),
                pl.BlockSpec((1, Vout), lambda b, tok: (0, 0)),
            ],
            out_specs=pl.BlockSpec((BB, S, Vout), lambda b, tok: (b, 0, 0)),
            scratch_shapes=[pltpu.VMEM((BB * S, D), jnp.float32)],
        ),
        compiler_params=pltpu.CompilerParams(dimension_semantics=("parallel",)),
    )(tokens, table_shift, spiral_pad, w1b, w2b, w3b,
      ln_w, ln_b, woutb, b_out)


# speedup vs baseline: 1.1518x; 1.0036x over previous
import math

import jax
import jax.numpy as jnp
from jax import lax
from jax.experimental import pallas as pl
from jax.experimental.pallas import tpu as pltpu

TEMPERATURE = 0.5
SPARSITY_THRESHOLD = 0.01
EDGE_WEIGHT = 0.1
LN_EPS = 1e-5


def _layer_norm(x, w, b):
    mean = jnp.mean(x, axis=-1, keepdims=True)
    m2 = jnp.mean(x * x, axis=-1, keepdims=True)
    var = m2 - mean * mean
    return (x - mean) * lax.rsqrt(var + LN_EPS) * w + b


def _gnn_chain(e, w1_ref, w2_ref, w3_ref, lnw, lnb, wout_ref, bout):
    BB, S, D = e.shape
    H = w1_ref.shape[1]
    V = wout_ref.shape[1]
    N = BB * S
    bf16 = jnp.bfloat16

    nsq = jnp.sum(e * e, axis=-1, keepdims=True)
    en = e * lax.rsqrt(jnp.maximum(nsq, 1e-12))

    dots = jnp.einsum('bsd,btd->bst', en, en,
                      preferred_element_type=jnp.float32)
    sq = 2.0 - 2.0 * dots
    dist = jnp.sqrt(jnp.maximum(sq, 1e-12))
    row = lax.broadcasted_iota(jnp.int32, (BB, S, S), 1)
    col = lax.broadcasted_iota(jnp.int32, (BB, S, S), 2)
    off_diag = row != col
    dist = jnp.where(off_diag, dist, 0.0)
    p = jnp.exp(dist * (-1.0 / TEMPERATURE))
    denom = jnp.sum(p, axis=-1, keepdims=True)
    attn = p * pl.reciprocal(denom, approx=True)

    A = jnp.where((attn > SPARSITY_THRESHOLD) & off_diag,
                  attn * EDGE_WEIGHT, 0.0)
    At = jnp.transpose(A, (0, 2, 1)).astype(bf16)

    def message_pass(x_flat, w_ref):
        h = jnp.dot(x_flat.astype(bf16), w_ref[...],
                    preferred_element_type=jnp.float32)
        h3 = h.reshape(BB, S, h.shape[-1]).astype(bf16)
        agg = jnp.einsum('bds,bsh->bdh', At, h3,
                         preferred_element_type=jnp.float32)
        return agg.reshape(N, h.shape[-1])

    x = en.reshape(N, D)
    x = _layer_norm(message_pass(x, w1_ref), lnw, lnb)
    x = _layer_norm(x + message_pass(x, w2_ref), lnw, lnb)
    x = _layer_norm(x + message_pass(x, w3_ref), lnw, lnb)

    logits = jnp.dot(x.astype(bf16), wout_ref[...],
                     preferred_element_type=jnp.float32) + bout
    return logits.reshape(BB, S, V)


def _fused_gnn_kernel(tok_ref, table_ref, spiral_ref, w1_ref, w2_ref, w3_ref,
                      lnw_ref, lnb_ref, wout_ref, bout_ref, out_ref,
                      grow_ref):
    BB, S, Vout = out_ref.shape
    D = spiral_ref.shape[2]
    blk = pl.program_id(0)
    nb = pl.num_programs(0)
    iota8 = lax.broadcasted_iota(jnp.int32, (8, D), 0)

    def gather_block(bidx, par):
        for b in range(BB):
            for k in range(S // 8):
                tile = spiral_ref[0, k * 8:k * 8 + 8, :]
                for j in range(8):
                    idx = tok_ref[bidx * BB + b, k * 8 + j]
                    r = table_ref[idx]
                    tile = jnp.where(iota8 == j, r + tile, tile)
                base = b * S + k * 8
                grow_ref[par, base:base + 8, :] = tile

    @pl.when(blk == 0)
    def _():
        gather_block(0, 0)

    p_cur = jax.lax.rem(blk, 2)
    e = grow_ref[p_cur].reshape(BB, S, D)
    gather_block(jnp.minimum(blk + 1, nb - 1), jax.lax.rem(blk + 1, 2))

    lnw = lnw_ref[...]
    lnb = lnb_ref[...]
    bout = bout_ref[...]
    out_ref[...] = _gnn_chain(e, w1_ref, w2_ref, w3_ref,
                              lnw, lnb, wout_ref, bout)


def kernel(tokens, token_embedding, w1, w2, w3, ln_w, ln_b, w_out, b_out):
    B, S = tokens.shape
    V, D = token_embedding.shape
    H = w1.shape[1]
    Vout = w_out.shape[1]
    max_seq_len = 512
    num_batch_blocks = 32
    BB = B // num_batch_blocks

    pos = jnp.arange(S, dtype=jnp.float32)
    thetas = 2.0 * math.pi * (pos / max_seq_len)
    rs = 0.3 + 0.6 * (pos / max(1, max_seq_len - 1))
    spiral = jnp.stack([rs * jnp.cos(thetas), rs * jnp.sin(thetas)], axis=-1)
    spiral_pad = jnp.concatenate(
        [spiral, jnp.zeros((S, D - 2), jnp.float32)], axis=-1)[None]
    table_shift = jnp.concatenate(
        [jnp.zeros((V, 2), jnp.float32), token_embedding[:, : D - 2]],
        axis=-1).reshape(V, 1, D)

    w1b = w1.astype(jnp.bfloat16)
    w2b = w2.astype(jnp.bfloat16)
    w3b = w3.astype(jnp.bfloat16)
    woutb = w_out.astype(jnp.bfloat16)

    return pl.pallas_call(
        _fused_gnn_kernel,
        out_shape=jax.ShapeDtypeStruct((B, S, Vout), jnp.float32),
        grid_spec=pltpu.PrefetchScalarGridSpec(
            num_scalar_prefetch=1,
            grid=(num_batch_blocks,),
            in_specs=[
                pl.BlockSpec((V, 1, D), lambda b, tok: (0, 0, 0)),
                pl.BlockSpec((1, S, D), lambda b, tok: (0, 0, 0)),
                pl.BlockSpec((D, H), lambda b, tok: (0, 0)),
                pl.BlockSpec((H, H), lambda b, tok: (0, 0)),
                pl.BlockSpec((H, H), lambda b, tok: (0, 0)),
                pl.BlockSpec((1, H), lambda b, tok: (0, 0)),
                pl.BlockSpec((1, H), lambda b, tok: (0, 0)),
                pl.BlockSpec((H, Vout), lambda b, tok: (0, 0)),
                pl.BlockSpec((1, Vout), lambda b, tok: (0, 0)),
            ],
            out_specs=pl.BlockSpec((BB, S, Vout), lambda b, tok: (b, 0, 0)),
            scratch_shapes=[pltpu.VMEM((2, BB * S, D), jnp.float32)],
        ),
        compiler_params=pltpu.CompilerParams(dimension_semantics=("arbitrary",)),
    )(tokens, table_shift, spiral_pad, w1b, w2b, w3b,
      ln_w, ln_b, woutb, b_out)
